# baseline (speedup 1.0000x reference)
import dataclasses
import functools

import jax
import jax.numpy as jnp
from jax import lax
from jax.experimental import pallas as pl
from jax.experimental.pallas import tpu as pltpu
from jax.experimental.pallas import tpu_sc as plsc

TOP_K = 8


def _mean_body(x_ref, o_ref):
    s = x_ref.shape[1]
    o_ref[...] = (jnp.sum(x_ref[0], axis=0, keepdims=True) * (1.0 / s))[None]


def _l2n(x):
    sq = jnp.sum(x * x, axis=-1, keepdims=True)
    return x * lax.rsqrt(jnp.maximum(sq, 1e-12))


def _simtopk_body(xm_ref, pk_ref, sim_ref, idx_ref, keyn_ref, rs_ref):
    b = xm_ref.shape[0]
    p = pk_ref.shape[0]
    key_norm = _l2n(pk_ref[...])
    x_norm = _l2n(xm_ref[...])
    keyn_ref[...] = key_norm
    sim = lax.dot_general(
        x_norm, key_norm,
        dimension_numbers=(((1,), (1,)), ((), ())),
        preferred_element_type=jnp.float32,
    )
    sim_ref[...] = sim
    ids = lax.broadcasted_iota(jnp.int32, (b, p), 1)
    cur = sim
    total = jnp.zeros((b, 1), jnp.float32)
    for k in range(TOP_K):
        m = jnp.max(cur, axis=1, keepdims=True)
        cand = jnp.where(cur == m, ids, jnp.int32(2**30))
        j = jnp.min(cand, axis=1, keepdims=True)
        idx_ref[:, k:k + 1] = j
        total = total + m
        cur = jnp.where(ids == j, -jnp.inf, cur)
    rs_ref[...] = jnp.sum(total, axis=0, keepdims=True) * (1.0 / b)


def _sc_gather(l, p, length, d, b, k):
    nrow = l * b * k * length
    nkey = b * k
    nw = 32
    rpw = nrow // nw
    kpw = nkey // nw
    cw = 32
    nchunk = rpw // cw
    nbuf = 4
    mesh = plsc.VectorSubcoreMesh(core_axis_name="c", subcore_axis_name="s")

    @functools.partial(
        pl.kernel,
        mesh=mesh,
        out_type=[
            jax.ShapeDtypeStruct((nrow, d), jnp.float32),
            jax.ShapeDtypeStruct((nkey, d), jnp.float32),
        ],
        scratch_types=[
            pltpu.VMEM((rpw,), jnp.int32),
            pltpu.VMEM((nkey,), jnp.int32),
            pltpu.VMEM((cw, d), jnp.float32),
            pltpu.VMEM((cw, d), jnp.float32),
            pltpu.VMEM((cw, d), jnp.float32),
            pltpu.VMEM((cw, d), jnp.float32),
            pltpu.VMEM((kpw, d), jnp.float32),
            pltpu.SemaphoreType.DMA,
            pltpu.SemaphoreType.DMA,
            pltpu.SemaphoreType.DMA,
            pltpu.SemaphoreType.DMA,
            pltpu.SemaphoreType.DMA,
            pltpu.SemaphoreType.DMA,
            pltpu.SemaphoreType.DMA,
        ],
    )
    def gather_kernel(table_hbm, keyn_hbm, gidx_hbm, kidx_hbm,
                      out1_hbm, out2_hbm,
                      gidx_v, kidx_v, buf0, buf1, buf2, buf3, krows_v,
                      sg0, sg1, sg2, sg3, sw0, sw1, sk):
        wid = lax.axis_index("s") * 2 + lax.axis_index("c")
        base = wid * rpw
        bufs = (buf0, buf1, buf2, buf3)
        gsems = (sg0, sg1, sg2, sg3)
        wsems = (sw0, sw1)

        pltpu.sync_copy(gidx_hbm.at[pl.ds(base, rpw)], gidx_v)

        pltpu.sync_copy(kidx_hbm, kidx_v)
        hk = pltpu.async_copy(
            keyn_hbm.at[kidx_v.at[pl.ds(wid * kpw, kpw)]], krows_v, sk)

        def g_start(c):
            return pltpu.async_copy(
                table_hbm.at[gidx_v.at[pl.ds(c * cw, cw)]],
                bufs[c % nbuf], gsems[c % nbuf])

        def w_start(c):
            return pltpu.async_copy(
                bufs[c % nbuf], out1_hbm.at[pl.ds(base + c * cw, cw)],
                wsems[c % 2])

        hg = {t: g_start(t) for t in range(min(3, nchunk))}
        hw = {}
        for c in range(nchunk):
            hg.pop(c).wait()
            hw[c] = w_start(c)
            nxt = c + 3
            if nxt < nchunk:
                if c >= 1:
                    hw.pop(c - 1).wait()
                hg[nxt] = g_start(nxt)
        for c in sorted(hw):
            hw.pop(c).wait()

        hk.wait()
        pltpu.sync_copy(krows_v, out2_hbm.at[pl.ds(wid * kpw, kpw)])

    return gather_kernel


def kernel(x_embed, prompt, prompt_key):
    b, s, d = x_embed.shape
    l, p, length, d2 = prompt.shape
    k = TOP_K

    x_mean = pl.pallas_call(
        _mean_body,
        grid=(b,),
        in_specs=[pl.BlockSpec((1, s, d), lambda i: (i, 0, 0))],
        out_specs=pl.BlockSpec((1, 1, d), lambda i: (i, 0, 0)),
        out_shape=jax.ShapeDtypeStruct((b, 1, d), jnp.float32),
        compiler_params=pltpu.CompilerParams(
            dimension_semantics=("parallel",)),
    )(x_embed)
    x_mean = x_mean.reshape(b, d)

    sim, idx, key_norm, rs = pl.pallas_call(
        _simtopk_body,
        out_shape=[
            jax.ShapeDtypeStruct((b, p), jnp.float32),
            jax.ShapeDtypeStruct((b, k), jnp.int32),
            jax.ShapeDtypeStruct((p, d), jnp.float32),
            jax.ShapeDtypeStruct((1, 1), jnp.float32),
        ],
    )(x_mean, prompt_key)

    flat = idx.reshape(-1)
    table = jnp.transpose(prompt, (0, 2, 1, 3)).reshape(l * length * p, d)
    gidx = (idx[None, :, :, None]
            + (jnp.arange(l, dtype=jnp.int32) * length * p)[:, None, None, None]
            + (jnp.arange(length, dtype=jnp.int32) * p)[None, None, None, :]
            ).reshape(-1)
    out1, out2 = _sc_gather(l, p, length, d, b, k)(table, key_norm, gidx, flat)

    batched_prompt = out1.reshape(l, b, k * length, d)
    batched_key_norm = out2.reshape(b, k, d)
    reduce_sim = rs.reshape(())
    return (sim, idx, batched_prompt, batched_key_norm, reduce_sim)
